# baseline (speedup 1.0000x reference)
import functools

import jax
import jax.numpy as jnp
from jax import lax
from jax.experimental import pallas as pl
from jax.experimental.pallas import tpu as pltpu
from jax.experimental.pallas import tpu_sc as plsc

N_NODES = 10000
N_EDGES = 320000
C = 128

NUM_CORES = 2
NUM_SUBCORES = 16

EDGE_BLOCK = 32
E_PAD = 327680
CHUNK_BLOCKS = 8
IDX_ROWS = E_PAD // 128
CHUNKS_PER_TILE = IDX_ROWS // (NUM_SUBCORES * CHUNK_BLOCKS)
FIRES_PER_CHUNK = CHUNK_BLOCKS * 128 // EDGE_BLOCK

SRC_SPLIT = 5024
X_ROWS = SRC_SPLIT
X_PAD_ROWS = 2 * SRC_SPLIT
AGG_ROWS = 10048
DUMMY_DST = 10016
XCOPY_ROWS = X_ROWS // 4
ZCOPY_ROWS = AGG_ROWS // 8


def _sc_aggregate(x_pad, packed_blocks, zeros_init):
    mesh = plsc.VectorSubcoreMesh(core_axis_name="c", subcore_axis_name="s")

    @functools.partial(
        pl.kernel,
        out_type=jax.ShapeDtypeStruct((NUM_CORES, AGG_ROWS, C), jnp.float32),
        mesh=mesh,
        scratch_types=[
            pltpu.VMEM((CHUNK_BLOCKS, 128), jnp.int32),
            pltpu.VMEM((1, EDGE_BLOCK), jnp.int32),
            pltpu.VMEM((1, EDGE_BLOCK), jnp.int32),
            pltpu.VMEM((1, EDGE_BLOCK), jnp.int32),
            pltpu.VMEM((1, EDGE_BLOCK), jnp.int32),
            pltpu.VMEM((EDGE_BLOCK, C), jnp.float32),
            pltpu.VMEM((EDGE_BLOCK, C), jnp.float32),
            pltpu.VMEM_SHARED((X_ROWS, C), jnp.float32),
            pltpu.VMEM_SHARED((AGG_ROWS, C), jnp.float32),
            pltpu.SemaphoreType.DMA,
            pltpu.SemaphoreType.DMA,
            pltpu.SemaphoreType.DMA,
            pltpu.SemaphoreType.DMA,
        ],
    )
    def agg_kernel(x_hbm, pk_hbm, zeros_hbm, out_hbm,
                   pk_v, fs0, fs1, fd0, fd1, rows0, rows1,
                   x_sh, agg_sh, g0, g1, s0, s1):
        c = lax.axis_index("c")
        s = lax.axis_index("s")

        fire_s = (fs0, fs1)
        fire_d = (fd0, fd1)
        rows = (rows0, rows1)
        gsem = (g0, g1)
        ssem = (s0, s1)

        zeros_v = jnp.zeros((16,), jnp.int32)
        dummy_v = jnp.full((16,), DUMMY_DST, jnp.int32)

        @pl.when(s < 8)
        def _init_agg():
            pltpu.sync_copy(zeros_hbm,
                            agg_sh.at[pl.ds(s * ZCOPY_ROWS, ZCOPY_ROWS)])

        @pl.when(s < 4)
        def _init_x():
            pltpu.sync_copy(
                x_hbm.at[pl.ds(c * X_ROWS + s * XCOPY_ROWS, XCOPY_ROWS)],
                x_sh.at[pl.ds(s * XCOPY_ROWS, XCOPY_ROWS)])
        plsc.subcore_barrier()

        def scan(src_lo):
            def chunk_body(ch, carry):
                blk0 = (s * (CHUNKS_PER_TILE * CHUNK_BLOCKS)
                        + ch * CHUNK_BLOCKS)
                pltpu.sync_copy(pk_hbm.at[pl.ds(blk0, CHUNK_BLOCKS)], pk_v)
                not_first = ch > 0

                for f in range(FIRES_PER_CHUNK):
                    b = f % 2
                    nb = 1 - b
                    if f >= 2:
                        pltpu.make_async_copy(
                            rows[b], agg_sh.at[fire_d[b].at[0]],
                            ssem[b]).wait()
                    else:
                        @pl.when(not_first)
                        def _drain_prev():
                            pltpu.make_async_copy(
                                rows[b], agg_sh.at[fire_d[b].at[0]],
                                ssem[b]).wait()

                    for q in range(2):
                        gi = f * 2 + q
                        r, gcol = gi // 8, (gi % 8) * 16
                        pk16 = pk_v[r, pl.ds(gcol, 16)]
                        src16 = lax.shift_right_logical(pk16, 14)
                        dst16 = jnp.bitwise_and(pk16, 16383)
                        if src_lo == 0:
                            m = src16 < SRC_SPLIT
                            srel = src16
                        else:
                            m = src16 >= SRC_SPLIT
                            srel = src16 - SRC_SPLIT
                        fire_s[b][0, pl.ds(q * 16, 16)] = jnp.where(
                            m, srel, zeros_v)
                        fire_d[b][0, pl.ds(q * 16, 16)] = jnp.where(
                            m, dst16, dummy_v)

                    pltpu.async_copy(x_sh.at[fire_s[b].at[0]], rows[b],
                                     gsem[b])
                    if f >= 1:
                        pltpu.make_async_copy(x_sh.at[fire_s[nb].at[0]],
                                              rows[nb], gsem[nb]).wait()
                        pltpu.async_copy(rows[nb],
                                         agg_sh.at[fire_d[nb].at[0]],
                                         ssem[nb], add=True)
                    else:
                        @pl.when(not_first)
                        def _scatter_prev():
                            pltpu.make_async_copy(x_sh.at[fire_s[nb].at[0]],
                                                  rows[nb],
                                                  gsem[nb]).wait()
                            pltpu.async_copy(rows[nb],
                                             agg_sh.at[fire_d[nb].at[0]],
                                             ssem[nb], add=True)
                return carry

            lax.fori_loop(0, CHUNKS_PER_TILE, chunk_body, jnp.int32(0))
            pltpu.make_async_copy(x_sh.at[fire_s[1].at[0]], rows1,
                                  g1).wait()
            pltpu.async_copy(rows1, agg_sh.at[fire_d[1].at[0]], s1,
                             add=True)
            pltpu.make_async_copy(rows0, agg_sh.at[fire_d[0].at[0]],
                                  s0).wait()
            pltpu.make_async_copy(rows1, agg_sh.at[fire_d[1].at[0]],
                                  s1).wait()

        pl.when(c == 0)(lambda: scan(0))
        pl.when(c == 1)(lambda: scan(SRC_SPLIT))
        plsc.subcore_barrier()

        @pl.when(s < 8)
        def _copy_out():
            pltpu.sync_copy(agg_sh.at[pl.ds(s * ZCOPY_ROWS, ZCOPY_ROWS)],
                            out_hbm.at[c, pl.ds(s * ZCOPY_ROWS, ZCOPY_ROWS)])

    return agg_kernel(x_pad, packed_blocks, zeros_init)


def _tc_tail_body(p_ref, x_ref, wrel_ref, wroot_ref, brel_ref,
                  gamma_ref, beta_ref, out_ref):
    agg = p_ref[0] + p_ref[1]
    xb = x_ref[...]
    h = (jnp.dot(agg, wrel_ref[...], preferred_element_type=jnp.float32)
         + jnp.dot(xb, wroot_ref[...], preferred_element_type=jnp.float32)
         + brel_ref[...])
    mean = jnp.mean(h, axis=-1, keepdims=True)
    var = jnp.mean((h - mean) * (h - mean), axis=-1, keepdims=True)
    hn = (h - mean) * lax.rsqrt(var + 1e-5) * gamma_ref[...] + beta_ref[...]
    out_ref[...] = jnp.maximum(hn, 0.0) + xb


ROW_BLOCK = 1000


def _tc_tail(partials, x, w_rel, w_root, b_rel, gamma, beta):
    grid = (N_NODES // ROW_BLOCK,)
    return pl.pallas_call(
        _tc_tail_body,
        out_shape=jax.ShapeDtypeStruct((N_NODES, C), jnp.float32),
        grid=grid,
        in_specs=[
            pl.BlockSpec((NUM_CORES, ROW_BLOCK, C), lambda i: (0, i, 0)),
            pl.BlockSpec((ROW_BLOCK, C), lambda i: (i, 0)),
            pl.BlockSpec((C, C), lambda i: (0, 0)),
            pl.BlockSpec((C, C), lambda i: (0, 0)),
            pl.BlockSpec((1, C), lambda i: (0, 0)),
            pl.BlockSpec((1, C), lambda i: (0, 0)),
            pl.BlockSpec((1, C), lambda i: (0, 0)),
        ],
        out_specs=pl.BlockSpec((ROW_BLOCK, C), lambda i: (i, 0)),
    )(partials, x, w_rel, w_root, b_rel, gamma, beta)


@jax.jit
def _run(x, edge_index, w_rel, b_rel, w_root, gamma, beta):
    src = edge_index[0]
    dst = edge_index[1]
    pad = E_PAD - N_EDGES
    src_p = jnp.concatenate([src, jnp.zeros((pad,), jnp.int32)])
    dst_p = jnp.concatenate([dst, jnp.full((pad,), DUMMY_DST, jnp.int32)])
    packed_blocks = (src_p * 16384 + dst_p).reshape(IDX_ROWS, 128)
    x_pad = jnp.concatenate(
        [x, jnp.zeros((X_PAD_ROWS - N_NODES, C), jnp.float32)])
    zeros_init = jnp.zeros((ZCOPY_ROWS, C), jnp.float32)

    partials = _sc_aggregate(x_pad, packed_blocks, zeros_init)
    return _tc_tail(partials, x, w_rel, w_root,
                    b_rel.reshape(1, C), gamma.reshape(1, C),
                    beta.reshape(1, C))


def kernel(x, edge_index, batch_size, W_rel, b_rel, W_root, gamma, beta):
    del batch_size
    return _run(x, edge_index, W_rel, b_rel, W_root, gamma, beta)


# speedup vs baseline: 1.0070x; 1.0070x over previous
import functools

import jax
import jax.numpy as jnp
from jax import lax
from jax.experimental import pallas as pl
from jax.experimental.pallas import tpu as pltpu
from jax.experimental.pallas import tpu_sc as plsc

N_NODES = 10000
N_EDGES = 320000
C = 128

NUM_CORES = 2
NUM_SUBCORES = 16

EDGE_BLOCK = 32
E_PAD = 327680
CHUNK_BLOCKS = 8
IDX_ROWS = E_PAD // 128
CHUNKS_PER_TILE = IDX_ROWS // (NUM_SUBCORES * CHUNK_BLOCKS)
FIRES_PER_CHUNK = CHUNK_BLOCKS * 128 // EDGE_BLOCK

SRC_SPLIT = 5024
X_ROWS = SRC_SPLIT
X_PAD_ROWS = 2 * SRC_SPLIT
AGG_ROWS = 10016
DUMMY_DST = 10000
X1_OFF = N_NODES - X_ROWS
XCOPY_ROWS = X_ROWS // 4
ZCOPY_ROWS = AGG_ROWS // 4


def _sc_aggregate(x_pad, packed_blocks, zeros_init):
    mesh = plsc.VectorSubcoreMesh(core_axis_name="c", subcore_axis_name="s")

    @functools.partial(
        pl.kernel,
        out_type=jax.ShapeDtypeStruct((NUM_CORES, AGG_ROWS, C), jnp.float32),
        mesh=mesh,
        scratch_types=[
            pltpu.VMEM((CHUNK_BLOCKS, 128), jnp.int32),
            pltpu.VMEM((CHUNK_BLOCKS, 128), jnp.int32),
            pltpu.VMEM((1, EDGE_BLOCK), jnp.int32),
            pltpu.VMEM((1, EDGE_BLOCK), jnp.int32),
            pltpu.VMEM((1, EDGE_BLOCK), jnp.int32),
            pltpu.VMEM((1, EDGE_BLOCK), jnp.int32),
            pltpu.VMEM((EDGE_BLOCK, C), jnp.float32),
            pltpu.VMEM((EDGE_BLOCK, C), jnp.float32),
            pltpu.VMEM_SHARED((X_ROWS, C), jnp.float32),
            pltpu.VMEM_SHARED((AGG_ROWS, C), jnp.float32),
            pltpu.SemaphoreType.DMA,
            pltpu.SemaphoreType.DMA,
            pltpu.SemaphoreType.DMA,
            pltpu.SemaphoreType.DMA,
            pltpu.SemaphoreType.DMA,
        ],
    )
    def agg_kernel(x_hbm, pk_hbm, zeros_hbm, out_hbm,
                   pk0, pk1, fs0, fs1, fd0, fd1, rows0, rows1,
                   x_sh, agg_sh, g0, g1, s0, s1, pks):
        c = lax.axis_index("c")
        s = lax.axis_index("s")

        fire_s = (fs0, fs1)
        fire_d = (fd0, fd1)
        rows = (rows0, rows1)
        gsem = (g0, g1)
        ssem = (s0, s1)

        zeros_v = jnp.zeros((16,), jnp.int32)
        dummy_v = jnp.full((16,), DUMMY_DST, jnp.int32)

        @pl.when(s < 4)
        def _init_agg():
            pltpu.sync_copy(zeros_hbm,
                            agg_sh.at[pl.ds(s * ZCOPY_ROWS, ZCOPY_ROWS)])

        @pl.when(s < 4)
        def _init_x():
            pltpu.sync_copy(
                x_hbm.at[pl.ds(c * X1_OFF + s * XCOPY_ROWS, XCOPY_ROWS)],
                x_sh.at[pl.ds(s * XCOPY_ROWS, XCOPY_ROWS)])
        plsc.subcore_barrier()

        def scan(src_lo):
            pk = (pk0, pk1)
            blk_base = s * (CHUNKS_PER_TILE * CHUNK_BLOCKS)

            def chunk_body(ch, pk_v, not_first, not_last):
                for f in range(FIRES_PER_CHUNK):
                    b = f % 2
                    nb = 1 - b
                    if f >= 2:
                        pltpu.make_async_copy(
                            rows[b], agg_sh.at[fire_d[b].at[0]],
                            ssem[b]).wait()
                    else:
                        @pl.when(not_first)
                        def _drain_prev():
                            pltpu.make_async_copy(
                                rows[b], agg_sh.at[fire_d[b].at[0]],
                                ssem[b]).wait()

                    for q in range(2):
                        gi = f * 2 + q
                        r, gcol = gi // 8, (gi % 8) * 16
                        pk16 = pk_v[r, pl.ds(gcol, 16)]
                        src16 = lax.shift_right_logical(pk16, 14)
                        dst16 = jnp.bitwise_and(pk16, 16383)
                        if src_lo == 0:
                            m = src16 < SRC_SPLIT
                            srel = src16
                        else:
                            m = src16 >= SRC_SPLIT
                            srel = src16 - X1_OFF
                        fire_s[b][0, pl.ds(q * 16, 16)] = jnp.where(
                            m, srel, zeros_v)
                        fire_d[b][0, pl.ds(q * 16, 16)] = jnp.where(
                            m, dst16, dummy_v)

                    pltpu.async_copy(x_sh.at[fire_s[b].at[0]], rows[b],
                                     gsem[b])
                    if f >= 1:
                        pltpu.make_async_copy(x_sh.at[fire_s[nb].at[0]],
                                              rows[nb], gsem[nb]).wait()
                        pltpu.async_copy(rows[nb],
                                         agg_sh.at[fire_d[nb].at[0]],
                                         ssem[nb], add=True)
                    else:
                        @pl.when(not_first)
                        def _scatter_prev():
                            pltpu.make_async_copy(x_sh.at[fire_s[nb].at[0]],
                                                  rows[nb],
                                                  gsem[nb]).wait()
                            pltpu.async_copy(rows[nb],
                                             agg_sh.at[fire_d[nb].at[0]],
                                             ssem[nb], add=True)

            pltpu.sync_copy(pk_hbm.at[pl.ds(blk_base, CHUNK_BLOCKS)], pk0)

            def two_chunks(i, carry):
                ch = i * 2
                for k in range(2):
                    chk = ch + k
                    pb = k
                    npb = 1 - k
                    @pl.when(chk > 0)
                    def _wait_idx():
                        pltpu.make_async_copy(
                            pk_hbm.at[pl.ds(blk_base + chk * CHUNK_BLOCKS,
                                            CHUNK_BLOCKS)],
                            pk[pb], pks).wait()

                    @pl.when(chk < CHUNKS_PER_TILE - 1)
                    def _prefetch():
                        pltpu.async_copy(
                            pk_hbm.at[pl.ds(
                                blk_base + (chk + 1) * CHUNK_BLOCKS,
                                CHUNK_BLOCKS)],
                            pk[npb], pks)
                    chunk_body(chk, pk[pb], chk > 0,
                               chk < CHUNKS_PER_TILE - 1)
                return carry

            lax.fori_loop(0, CHUNKS_PER_TILE // 2, two_chunks, jnp.int32(0))
            pltpu.make_async_copy(x_sh.at[fire_s[1].at[0]], rows1,
                                  g1).wait()
            pltpu.async_copy(rows1, agg_sh.at[fire_d[1].at[0]], s1,
                             add=True)
            pltpu.make_async_copy(rows0, agg_sh.at[fire_d[0].at[0]],
                                  s0).wait()
            pltpu.make_async_copy(rows1, agg_sh.at[fire_d[1].at[0]],
                                  s1).wait()

        pl.when(c == 0)(lambda: scan(0))
        pl.when(c == 1)(lambda: scan(X1_OFF))
        plsc.subcore_barrier()

        @pl.when(s < 4)
        def _copy_out():
            pltpu.sync_copy(agg_sh.at[pl.ds(s * ZCOPY_ROWS, ZCOPY_ROWS)],
                            out_hbm.at[c, pl.ds(s * ZCOPY_ROWS, ZCOPY_ROWS)])

    return agg_kernel(x_pad, packed_blocks, zeros_init)


def _tc_tail_body(p_ref, x_ref, wrel_ref, wroot_ref, brel_ref,
                  gamma_ref, beta_ref, out_ref):
    agg = p_ref[0] + p_ref[1]
    xb = x_ref[...]
    h = (jnp.dot(agg, wrel_ref[...], preferred_element_type=jnp.float32)
         + jnp.dot(xb, wroot_ref[...], preferred_element_type=jnp.float32)
         + brel_ref[...])
    mean = jnp.mean(h, axis=-1, keepdims=True)
    var = jnp.mean((h - mean) * (h - mean), axis=-1, keepdims=True)
    hn = (h - mean) * lax.rsqrt(var + 1e-5) * gamma_ref[...] + beta_ref[...]
    out_ref[...] = jnp.maximum(hn, 0.0) + xb


ROW_BLOCK = 1000


def _tc_tail(partials, x, w_rel, w_root, b_rel, gamma, beta):
    grid = (N_NODES // ROW_BLOCK,)
    return pl.pallas_call(
        _tc_tail_body,
        out_shape=jax.ShapeDtypeStruct((N_NODES, C), jnp.float32),
        grid=grid,
        in_specs=[
            pl.BlockSpec((NUM_CORES, ROW_BLOCK, C), lambda i: (0, i, 0)),
            pl.BlockSpec((ROW_BLOCK, C), lambda i: (i, 0)),
            pl.BlockSpec((C, C), lambda i: (0, 0)),
            pl.BlockSpec((C, C), lambda i: (0, 0)),
            pl.BlockSpec((1, C), lambda i: (0, 0)),
            pl.BlockSpec((1, C), lambda i: (0, 0)),
            pl.BlockSpec((1, C), lambda i: (0, 0)),
        ],
        out_specs=pl.BlockSpec((ROW_BLOCK, C), lambda i: (i, 0)),
    )(partials, x, w_rel, w_root, b_rel, gamma, beta)


@jax.jit
def _run(x, edge_index, w_rel, b_rel, w_root, gamma, beta):
    src = edge_index[0]
    dst = edge_index[1]
    pad = E_PAD - N_EDGES
    src_p = jnp.concatenate([src, jnp.zeros((pad,), jnp.int32)])
    dst_p = jnp.concatenate([dst, jnp.full((pad,), DUMMY_DST, jnp.int32)])
    packed_blocks = (src_p * 16384 + dst_p).reshape(IDX_ROWS, 128)
    zeros_init = jnp.zeros((ZCOPY_ROWS, C), jnp.float32)

    partials = _sc_aggregate(x, packed_blocks, zeros_init)
    return _tc_tail(partials, x, w_rel, w_root,
                    b_rel.reshape(1, C), gamma.reshape(1, C),
                    beta.reshape(1, C))


def kernel(x, edge_index, batch_size, W_rel, b_rel, W_root, gamma, beta):
    del batch_size
    return _run(x, edge_index, W_rel, b_rel, W_root, gamma, beta)
